# baseline (speedup 1.0000x reference)
import functools

import jax
import jax.numpy as jnp
from jax import lax
from jax.experimental import pallas as pl
from jax.experimental.pallas import tpu as pltpu
from jax.experimental.pallas import tpu_sc as plsc

N = 2048
DIM = 768
E = 8
TOPK = 2
HID = 2048
S = N * TOPK
TM = 512
N_TILES = -(-(S + E * (TM - 1)) // TM)
S_PAD = N_TILES * TM

SC_CORES = 2
SC_SUBCORES = 16
NW = SC_CORES * SC_SUBCORES
TOK_CH = N // NW


def _sc_dispatch_body(x_hbm, post_hbm, out_hbm, rows_v, idx_v, sem):
    wid = lax.axis_index("s") * SC_CORES + lax.axis_index("c")
    tok0 = wid * TOK_CH
    pltpu.sync_copy(x_hbm.at[pl.ds(tok0, TOK_CH)], rows_v)
    pltpu.sync_copy(post_hbm.at[0, pl.ds(tok0, TOK_CH)], idx_v.at[0])
    pltpu.sync_copy(post_hbm.at[1, pl.ds(tok0, TOK_CH)], idx_v.at[1])
    a = pltpu.async_copy(rows_v, out_hbm.at[idx_v.at[0]], sem)
    b = pltpu.async_copy(rows_v, out_hbm.at[idx_v.at[1]], sem)
    a.wait()
    b.wait()


def _sc_dispatch(x_flat, pos_t):
    mesh = plsc.VectorSubcoreMesh(
        core_axis_name="c", subcore_axis_name="s",
        num_cores=SC_CORES, num_subcores=SC_SUBCORES)
    kern = pl.kernel(
        _sc_dispatch_body,
        out_type=jax.ShapeDtypeStruct((S_PAD, DIM), jnp.float32),
        mesh=mesh,
        scratch_types=[
            pltpu.VMEM((TOK_CH, DIM), jnp.float32),
            pltpu.VMEM((2, TOK_CH), jnp.int32),
            pltpu.SemaphoreType.DMA,
        ],
    )
    return kern(x_flat, pos_t)


def _moe_tile(te_ref, nact_ref, x_ref, w1_ref, w3_ref, w2_ref, y_ref):
    m = pl.program_id(0)

    @pl.when(m < nact_ref[0])
    def _():
        x = x_ref[...].astype(jnp.bfloat16)
        w1 = w1_ref[0].astype(jnp.bfloat16)
        w3 = w3_ref[0].astype(jnp.bfloat16)
        w2 = w2_ref[0].astype(jnp.bfloat16)
        dn = (((1,), (1,)), ((), ()))
        h1 = lax.dot_general(x, w1, dn, preferred_element_type=jnp.float32)
        h3 = lax.dot_general(x, w3, dn, preferred_element_type=jnp.float32)
        h = (h1 * jax.nn.sigmoid(h1)) * h3
        y_ref[...] = lax.dot_general(h.astype(jnp.bfloat16), w2, dn,
                                     preferred_element_type=jnp.float32)


def _grouped_ffn(x_sorted, W1, W3, W2, tile_expert, n_active):
    clamp = lambda m, na: jnp.minimum(m, na[0] - 1)
    grid_spec = pltpu.PrefetchScalarGridSpec(
        num_scalar_prefetch=2,
        grid=(N_TILES,),
        in_specs=[
            pl.BlockSpec((TM, DIM), lambda m, te, na: (clamp(m, na), 0)),
            pl.BlockSpec((1, HID, DIM), lambda m, te, na: (te[m], 0, 0)),
            pl.BlockSpec((1, HID, DIM), lambda m, te, na: (te[m], 0, 0)),
            pl.BlockSpec((1, DIM, HID), lambda m, te, na: (te[m], 0, 0)),
        ],
        out_specs=pl.BlockSpec((TM, DIM), lambda m, te, na: (clamp(m, na), 0)),
    )
    return pl.pallas_call(
        _moe_tile,
        grid_spec=grid_spec,
        out_shape=jax.ShapeDtypeStruct((S_PAD, DIM), jnp.float32),
        compiler_params=pltpu.CompilerParams(
            dimension_semantics=("arbitrary",),
            vmem_limit_bytes=100 * 1024 * 1024),
    )(tile_expert, n_active, x_sorted, W1, W3, W2)


def kernel(x, gate_w, W1, W2, W3):
    bsz, seqlen, dim = x.shape
    x_flat = x.reshape(-1, dim)

    gate_logits = x_flat @ gate_w.T
    lt = gate_logits.T
    eidx = jnp.arange(E, dtype=jnp.int32)
    e0 = jnp.argmax(lt, axis=0).astype(jnp.int32)
    v0 = jnp.max(lt, axis=0)
    masked = jnp.where(eidx[:, None] == e0[None, :], -jnp.inf, lt)
    e1 = jnp.argmax(masked, axis=0).astype(jnp.int32)
    v1 = jnp.max(masked, axis=0)
    z = jnp.exp(v1 - v0)
    w0 = 1.0 / (1.0 + z)
    w1r = z / (1.0 + z)

    ef = jnp.stack([e0, e1], axis=0)
    oh = (ef[None, :, :] == eidx[:, None, None]).astype(jnp.int32)
    cs = jnp.cumsum(oh.reshape(E, S), axis=1).reshape(E, TOPK, N)
    counts = cs.reshape(E, S)[:, -1]
    rank = jnp.sum(jnp.where(oh != 0, cs, 0), axis=0) - 1
    padded_counts = ((counts + TM - 1) // TM) * TM
    padded_offsets = jnp.concatenate(
        [jnp.zeros(1, jnp.int32),
         jnp.cumsum(padded_counts).astype(jnp.int32)])
    off = jnp.sum(jnp.where(oh != 0, padded_offsets[:E, None, None], 0), axis=0)
    pos_t = (off + rank).astype(jnp.int32)

    total_padded = padded_offsets[E]
    n_active = (total_padded // TM).astype(jnp.int32).reshape(1)
    tile_starts = jnp.arange(N_TILES, dtype=jnp.int32) * TM
    tile_expert = jnp.clip(
        jnp.sum(padded_offsets[1:E][None, :] <= tile_starts[:, None], axis=1),
        0, E - 1).astype(jnp.int32)
    tile_expert = jnp.where(tile_starts < total_padded, tile_expert,
                            tile_expert[jnp.maximum(n_active[0] - 1, 0)])

    x_sorted = _sc_dispatch(x_flat, pos_t)

    y = _grouped_ffn(x_sorted, W1, W3, W2, tile_expert, n_active)

    out = (w0[:, None] * jnp.take(y, pos_t[0], axis=0)
           + w1r[:, None] * jnp.take(y, pos_t[1], axis=0))
    return out.astype(x.dtype).reshape(bsz, seqlen, dim)


# speedup vs baseline: 1.1952x; 1.1952x over previous
import functools

import jax
import jax.numpy as jnp
from jax import lax
from jax.experimental import pallas as pl
from jax.experimental.pallas import tpu as pltpu
from jax.experimental.pallas import tpu_sc as plsc

N = 2048
DIM = 768
E = 8
TOPK = 2
HID = 2048
S = N * TOPK
TM = 576
N_TILES = -(-(S + E * (TM - 1)) // TM)
S_PAD = N_TILES * TM

SC_CORES = 2
SC_SUBCORES = 16
NW = SC_CORES * SC_SUBCORES
TOK_CH = N // NW


def _sc_dispatch_body(x_hbm, post_hbm, out_hbm, rows_v, idx_v, sem):
    wid = lax.axis_index("s") * SC_CORES + lax.axis_index("c")
    tok0 = wid * TOK_CH
    pltpu.sync_copy(x_hbm.at[pl.ds(tok0, TOK_CH)], rows_v)
    pltpu.sync_copy(post_hbm.at[0, pl.ds(tok0, TOK_CH)], idx_v.at[0])
    pltpu.sync_copy(post_hbm.at[1, pl.ds(tok0, TOK_CH)], idx_v.at[1])
    a = pltpu.async_copy(rows_v, out_hbm.at[idx_v.at[0]], sem)
    b = pltpu.async_copy(rows_v, out_hbm.at[idx_v.at[1]], sem)
    a.wait()
    b.wait()


def _sc_dispatch(x_flat, pos_t):
    mesh = plsc.VectorSubcoreMesh(
        core_axis_name="c", subcore_axis_name="s",
        num_cores=SC_CORES, num_subcores=SC_SUBCORES)
    kern = pl.kernel(
        _sc_dispatch_body,
        out_type=jax.ShapeDtypeStruct((S_PAD, DIM), jnp.float32),
        mesh=mesh,
        scratch_types=[
            pltpu.VMEM((TOK_CH, DIM), jnp.float32),
            pltpu.VMEM((2, TOK_CH), jnp.int32),
            pltpu.SemaphoreType.DMA,
        ],
    )
    return kern(x_flat, pos_t)


def _moe_tile(te_ref, nact_ref, x_ref, w1_ref, w3_ref, w2_ref, y_ref):
    m = pl.program_id(0)

    @pl.when(m < nact_ref[0])
    def _():
        x = x_ref[...].astype(jnp.bfloat16)
        w1 = w1_ref[0].astype(jnp.bfloat16)
        w3 = w3_ref[0].astype(jnp.bfloat16)
        w2 = w2_ref[0].astype(jnp.bfloat16)
        dn = (((1,), (1,)), ((), ()))
        h1 = lax.dot_general(x, w1, dn, preferred_element_type=jnp.float32)
        h3 = lax.dot_general(x, w3, dn, preferred_element_type=jnp.float32)
        h = (h1 * jax.nn.sigmoid(h1)) * h3
        y_ref[...] = lax.dot_general(h.astype(jnp.bfloat16), w2, dn,
                                     preferred_element_type=jnp.float32)


def _grouped_ffn(x_sorted, W1, W3, W2, tile_expert, n_active):
    clamp = lambda m, na: jnp.minimum(m, na[0] - 1)
    grid_spec = pltpu.PrefetchScalarGridSpec(
        num_scalar_prefetch=2,
        grid=(N_TILES,),
        in_specs=[
            pl.BlockSpec((TM, DIM), lambda m, te, na: (clamp(m, na), 0)),
            pl.BlockSpec((1, HID, DIM), lambda m, te, na: (te[m], 0, 0)),
            pl.BlockSpec((1, HID, DIM), lambda m, te, na: (te[m], 0, 0)),
            pl.BlockSpec((1, DIM, HID), lambda m, te, na: (te[m], 0, 0)),
        ],
        out_specs=pl.BlockSpec((TM, DIM), lambda m, te, na: (clamp(m, na), 0)),
    )
    return pl.pallas_call(
        _moe_tile,
        grid_spec=grid_spec,
        out_shape=jax.ShapeDtypeStruct((S_PAD, DIM), jnp.float32),
        compiler_params=pltpu.CompilerParams(
            dimension_semantics=("arbitrary",),
            vmem_limit_bytes=100 * 1024 * 1024),
    )(tile_expert, n_active, x_sorted, W1, W3, W2)


def kernel(x, gate_w, W1, W2, W3):
    bsz, seqlen, dim = x.shape
    x_flat = x.reshape(-1, dim)

    gate_logits = x_flat @ gate_w.T
    lt = gate_logits.T
    eidx = jnp.arange(E, dtype=jnp.int32)
    e0 = jnp.argmax(lt, axis=0).astype(jnp.int32)
    v0 = jnp.max(lt, axis=0)
    masked = jnp.where(eidx[:, None] == e0[None, :], -jnp.inf, lt)
    e1 = jnp.argmax(masked, axis=0).astype(jnp.int32)
    v1 = jnp.max(masked, axis=0)
    z = jnp.exp(v1 - v0)
    w0 = 1.0 / (1.0 + z)
    w1r = z / (1.0 + z)

    ef = jnp.stack([e0, e1], axis=0)
    oh = (ef[None, :, :] == eidx[:, None, None]).astype(jnp.int32)
    cs = jnp.cumsum(oh.reshape(E, S), axis=1).reshape(E, TOPK, N)
    counts = cs.reshape(E, S)[:, -1]
    rank = jnp.sum(jnp.where(oh != 0, cs, 0), axis=0) - 1
    padded_counts = ((counts + TM - 1) // TM) * TM
    padded_offsets = jnp.concatenate(
        [jnp.zeros(1, jnp.int32),
         jnp.cumsum(padded_counts).astype(jnp.int32)])
    off = jnp.sum(jnp.where(oh != 0, padded_offsets[:E, None, None], 0), axis=0)
    pos_t = (off + rank).astype(jnp.int32)

    total_padded = padded_offsets[E]
    n_active = (total_padded // TM).astype(jnp.int32).reshape(1)
    tile_starts = jnp.arange(N_TILES, dtype=jnp.int32) * TM
    tile_expert = jnp.clip(
        jnp.sum(padded_offsets[1:E][None, :] <= tile_starts[:, None], axis=1),
        0, E - 1).astype(jnp.int32)
    tile_expert = jnp.where(tile_starts < total_padded, tile_expert,
                            tile_expert[jnp.maximum(n_active[0] - 1, 0)])

    x_sorted = _sc_dispatch(x_flat, pos_t)

    y = _grouped_ffn(x_sorted, W1, W3, W2, tile_expert, n_active)

    out = (w0[:, None] * jnp.take(y, pos_t[0], axis=0)
           + w1r[:, None] * jnp.take(y, pos_t[1], axis=0))
    return out.astype(x.dtype).reshape(bsz, seqlen, dim)


# speedup vs baseline: 1.2190x; 1.0199x over previous
import functools

import jax
import jax.numpy as jnp
from jax import lax
from jax.experimental import pallas as pl
from jax.experimental.pallas import tpu as pltpu
from jax.experimental.pallas import tpu_sc as plsc

N = 2048
DIM = 768
E = 8
TOPK = 2
HID = 2048
S = N * TOPK
TM = 576
N_TILES = -(-(S + E * (TM - 1)) // TM)
S_PAD = N_TILES * TM

SC_CORES = 2
SC_SUBCORES = 16
NW = SC_CORES * SC_SUBCORES
TOK_CH = N // NW


def _sc_dispatch_body(x_hbm, post_hbm, out_hbm, rows_v, idx_v, sem):
    wid = lax.axis_index("s") * SC_CORES + lax.axis_index("c")
    tok0 = wid * TOK_CH
    pltpu.sync_copy(x_hbm.at[pl.ds(tok0, TOK_CH)], rows_v)
    pltpu.sync_copy(post_hbm.at[0, pl.ds(tok0, TOK_CH)], idx_v.at[0])
    pltpu.sync_copy(post_hbm.at[1, pl.ds(tok0, TOK_CH)], idx_v.at[1])
    a = pltpu.async_copy(rows_v, out_hbm.at[idx_v.at[0]], sem)
    b = pltpu.async_copy(rows_v, out_hbm.at[idx_v.at[1]], sem)
    a.wait()
    b.wait()


def _sc_dispatch(x_flat, pos_t):
    mesh = plsc.VectorSubcoreMesh(
        core_axis_name="c", subcore_axis_name="s",
        num_cores=SC_CORES, num_subcores=SC_SUBCORES)
    kern = pl.kernel(
        _sc_dispatch_body,
        out_type=jax.ShapeDtypeStruct((S_PAD, DIM), jnp.float32),
        mesh=mesh,
        scratch_types=[
            pltpu.VMEM((TOK_CH, DIM), jnp.float32),
            pltpu.VMEM((2, TOK_CH), jnp.int32),
            pltpu.SemaphoreType.DMA,
        ],
    )
    return kern(x_flat, pos_t)


def _moe_tile(te_ref, nact_ref, x_ref, w1_ref, w3_ref, w2_ref, y_ref):
    m = pl.program_id(0)

    @pl.when(m < nact_ref[0])
    def _():
        x = x_ref[...].astype(jnp.bfloat16)
        w1 = w1_ref[0].astype(jnp.bfloat16)
        w3 = w3_ref[0].astype(jnp.bfloat16)
        w2 = w2_ref[0].astype(jnp.bfloat16)
        dn = (((1,), (1,)), ((), ()))
        h1 = lax.dot_general(x, w1, dn, preferred_element_type=jnp.float32)
        h3 = lax.dot_general(x, w3, dn, preferred_element_type=jnp.float32)
        h = (h1 * jax.nn.sigmoid(h1)) * h3
        y_ref[...] = lax.dot_general(h.astype(jnp.bfloat16), w2, dn,
                                     preferred_element_type=jnp.float32)


def _grouped_ffn(x_sorted, W1, W3, W2, tile_expert, n_active):
    clamp = lambda m, na: jnp.minimum(m, na[0] - 1)
    grid_spec = pltpu.PrefetchScalarGridSpec(
        num_scalar_prefetch=2,
        grid=(N_TILES,),
        in_specs=[
            pl.BlockSpec((TM, DIM), lambda m, te, na: (clamp(m, na), 0)),
            pl.BlockSpec((1, HID, DIM), lambda m, te, na: (te[m], 0, 0)),
            pl.BlockSpec((1, HID, DIM), lambda m, te, na: (te[m], 0, 0)),
            pl.BlockSpec((1, DIM, HID), lambda m, te, na: (te[m], 0, 0)),
        ],
        out_specs=pl.BlockSpec((TM, DIM), lambda m, te, na: (clamp(m, na), 0)),
    )
    return pl.pallas_call(
        _moe_tile,
        grid_spec=grid_spec,
        out_shape=jax.ShapeDtypeStruct((S_PAD, DIM), jnp.float32),
        compiler_params=pltpu.CompilerParams(
            dimension_semantics=("arbitrary",),
            vmem_limit_bytes=100 * 1024 * 1024),
    )(tile_expert, n_active, x_sorted, W1, W3, W2)


def kernel(x, gate_w, W1, W2, W3):
    bsz, seqlen, dim = x.shape
    x_flat = x.reshape(-1, dim)

    gate_logits = x_flat @ gate_w.T
    lt = gate_logits.T
    eidx = jnp.arange(E, dtype=jnp.int32)
    e0 = jnp.argmax(lt, axis=0).astype(jnp.int32)
    v0 = jnp.max(lt, axis=0)
    masked = jnp.where(eidx[:, None] == e0[None, :], -jnp.inf, lt)
    e1 = jnp.argmax(masked, axis=0).astype(jnp.int32)
    v1 = jnp.max(masked, axis=0)
    z = jnp.exp(v1 - v0)
    w0 = 1.0 / (1.0 + z)
    w1r = z / (1.0 + z)

    ef = jnp.stack([e0, e1], axis=0)
    oh = (ef[None, :, :] == eidx[:, None, None]).astype(jnp.int32)
    cs = jnp.cumsum(oh.reshape(E, S), axis=1).reshape(E, TOPK, N)
    counts = cs.reshape(E, S)[:, -1]
    rank = jnp.sum(jnp.where(oh != 0, cs, 0), axis=0) - 1
    padded_counts = ((counts + TM - 1) // TM) * TM
    padded_offsets = jnp.concatenate(
        [jnp.zeros(1, jnp.int32),
         jnp.cumsum(padded_counts).astype(jnp.int32)])
    off = jnp.sum(jnp.where(oh != 0, padded_offsets[:E, None, None], 0), axis=0)
    pos_t = (off + rank).astype(jnp.int32)

    total_padded = padded_offsets[E]
    n_active = (total_padded // TM).astype(jnp.int32).reshape(1)
    tile_starts = jnp.arange(N_TILES, dtype=jnp.int32) * TM
    tile_expert = jnp.clip(
        jnp.sum(padded_offsets[1:E][None, :] <= tile_starts[:, None], axis=1),
        0, E - 1).astype(jnp.int32)
    tile_expert = jnp.where(tile_starts < total_padded, tile_expert,
                            tile_expert[jnp.maximum(n_active[0] - 1, 0)])

    x_sorted = _sc_dispatch(x_flat, pos_t)

    y = _grouped_ffn(x_sorted, W1, W3, W2, tile_expert, n_active)

    g = jnp.take(y, pos_t.reshape(-1), axis=0)
    out = w0[:, None] * g[:N] + w1r[:, None] * g[N:]
    return out.astype(x.dtype).reshape(bsz, seqlen, dim)
